# baseline (speedup 1.0000x reference)
import jax
import jax.numpy as jnp
from jax.experimental import pallas as pl

_K = 2


def _fused_body(x_ref, wg_ref, w_ref, beta_ref, o_ref, loss_ref):
    B = x_ref.shape[0]
    E = wg_ref.shape[1]
    f32 = jnp.float32

    xm = jnp.mean(x_ref[...], axis=2)
    logits = jnp.dot(xm, wg_ref[...], preferred_element_type=f32)
    m = jnp.max(logits, axis=1, keepdims=True)
    ex = jnp.exp(logits - m)
    sm = ex / jnp.sum(ex, axis=1, keepdims=True)

    col = jax.lax.broadcasted_iota(jnp.int32, (B, E), 1)
    v1 = jnp.max(sm, axis=1, keepdims=True)
    i1 = jnp.min(jnp.where(sm == v1, col, E + 1), axis=1, keepdims=True)
    sm2 = jnp.where(col == i1, -jnp.inf, sm)
    v2 = jnp.max(sm2, axis=1, keepdims=True)
    i2 = jnp.min(jnp.where(sm2 == v2, col, E + 1), axis=1, keepdims=True)

    den = v1 + v2 + 1e-6
    g1 = v1 / den
    g2 = v2 / den

    dense = jnp.where(col == i1, g1, 0.0) + jnp.where(col == i2, g2, 0.0)
    imp = jnp.sum(dense, axis=0)
    load = jnp.sum((dense > 0.0).astype(f32), axis=0)

    def cv_sq(v):
        mu = jnp.mean(v)
        var = jnp.sum((v - mu) ** 2) / (E - 1)
        return var / (mu * mu + 1e-10)

    loss_ref[...] = jnp.reshape((cv_sq(imp) + cv_sq(load)) * 1e-2, (1, 1))

    brow = jax.lax.broadcasted_iota(jnp.int32, (B, 1), 0)
    bv = beta_ref[...]
    ri = jax.lax.broadcasted_iota(jnp.int32, bv.shape, 0)
    ci_ = jax.lax.broadcasted_iota(jnp.int32, bv.shape, 1)
    W = 16
    HW = x_ref.shape[2]
    lane = jax.lax.broadcasted_iota(jnp.int32, (1, HW), 1) % W

    def sample(b, carry):
        xt = jnp.tanh(x_ref[b])
        xt2 = xt * xt

        acc = jnp.zeros((o_ref.shape[1], HW), f32)
        for k in range(_K):
            iarr = i1 if k == 0 else i2
            garr = g1 if k == 0 else g2
            e = jnp.sum(jnp.where(brow == b, iarr, 0))
            gate = jnp.sum(jnp.where(brow == b, garr, 0.0))
            b2 = 2.25 * jnp.sum(jnp.where((ri == e) & (ci_ == 1), bv, 0.0))
            b3 = (300.0 / 9.0) * jnp.sum(
                jnp.where((ri == e) & (ci_ == 2), bv, 0.0))

            p2 = xt2 - b2
            p3 = xt * p2 - b3 * xt
            g = jnp.concatenate([jnp.ones_like(xt), xt, p2, p3], axis=0)
            g = g * jax.nn.sigmoid(g)

            CI = g.shape[0]
            padz = jnp.zeros((CI, 2 * W), dtype=g.dtype)
            gext = jnp.concatenate([padz, g, padz], axis=1).astype(jnp.bfloat16)

            acck = jnp.zeros((o_ref.shape[1], HW), f32)
            for j in range(9):
                dy, dx = j // 3, j % 3
                off = W * (dy - 1) + (dx - 1)
                s = jax.lax.slice(gext, (0, 2 * W + off),
                                  (CI, 2 * W + off + HW))
                if dx == 0:
                    s = jnp.where(lane != 0, s, jnp.bfloat16(0))
                elif dx == 2:
                    s = jnp.where(lane != W - 1, s, jnp.bfloat16(0))
                acck = acck + jax.lax.dot(
                    w_ref[e, j], s, preferred_element_type=f32)
            acc = acc + gate * acck

        o_ref[b] = acc
        return carry

    jax.lax.fori_loop(0, B, sample, 0)


def kernel(x, w_gate, poly_weights, beta_weights):
    B, C, H, W = x.shape
    E, O, CI, KH, KW = poly_weights.shape
    HW = H * W
    x2 = x.reshape(B, C, HW)
    pwt = jnp.transpose(poly_weights, (0, 3, 4, 1, 2)).reshape(
        E, KH * KW, O, CI).astype(jnp.bfloat16)

    y, loss = pl.pallas_call(
        _fused_body,
        out_shape=[
            jax.ShapeDtypeStruct((B, O, HW), jnp.float32),
            jax.ShapeDtypeStruct((1, 1), jnp.float32),
        ],
    )(x2, w_gate, pwt, beta_weights)

    return y.reshape(B, O, H, W), loss[0, 0]


# speedup vs baseline: 1.0093x; 1.0093x over previous
import jax
import jax.numpy as jnp
from jax.experimental import pallas as pl

_K = 2


def _fused_body(x_ref, wg_ref, w_ref, beta_ref, o_ref, loss_ref):
    B = x_ref.shape[0]
    E = wg_ref.shape[1]
    f32 = jnp.float32

    xm = jnp.mean(x_ref[...], axis=2)
    logits = jnp.dot(xm, wg_ref[...], preferred_element_type=f32)
    m = jnp.max(logits, axis=1, keepdims=True)
    ex = jnp.exp(logits - m)
    sm = ex / jnp.sum(ex, axis=1, keepdims=True)

    col = jax.lax.broadcasted_iota(jnp.int32, (B, E), 1)
    v1 = jnp.max(sm, axis=1, keepdims=True)
    i1 = jnp.min(jnp.where(sm == v1, col, E + 1), axis=1, keepdims=True)
    sm2 = jnp.where(col == i1, -jnp.inf, sm)
    v2 = jnp.max(sm2, axis=1, keepdims=True)
    i2 = jnp.min(jnp.where(sm2 == v2, col, E + 1), axis=1, keepdims=True)

    den = v1 + v2 + 1e-6
    g1 = v1 / den
    g2 = v2 / den

    dense = jnp.where(col == i1, g1, 0.0) + jnp.where(col == i2, g2, 0.0)
    imp = jnp.sum(dense, axis=0)
    load = jnp.sum((dense > 0.0).astype(f32), axis=0)

    def cv_sq(v):
        mu = jnp.mean(v)
        var = jnp.sum((v - mu) ** 2) / (E - 1)
        return var / (mu * mu + 1e-10)

    loss_ref[...] = jnp.reshape((cv_sq(imp) + cv_sq(load)) * 1e-2, (1, 1))

    brow = jax.lax.broadcasted_iota(jnp.int32, (B, 1), 0)
    bv = beta_ref[...]
    ri = jax.lax.broadcasted_iota(jnp.int32, bv.shape, 0)
    ci_ = jax.lax.broadcasted_iota(jnp.int32, bv.shape, 1)
    W = 16
    HW = x_ref.shape[2]
    lane320 = jax.lax.broadcasted_iota(jnp.int32, (1, HW + 4 * W), 1) % W

    def sample(b, carry):
        xt = jnp.tanh(x_ref[b])
        xt2 = xt * xt
        xts = xt * jax.nn.sigmoid(xt)

        acc = jnp.zeros((o_ref.shape[1], HW), f32)
        for k in range(_K):
            iarr = i1 if k == 0 else i2
            garr = g1 if k == 0 else g2
            e = jnp.sum(jnp.where(brow == b, iarr, 0))
            gate = jnp.sum(jnp.where(brow == b, garr, 0.0))
            b2 = 2.25 * jnp.sum(jnp.where((ri == e) & (ci_ == 1), bv, 0.0))
            b3 = (300.0 / 9.0) * jnp.sum(
                jnp.where((ri == e) & (ci_ == 2), bv, 0.0))

            p2 = xt2 - b2
            p3 = xt * p2 - b3 * xt
            g23 = jnp.concatenate([p2, p3], axis=0)
            g23 = g23 * jax.nn.sigmoid(g23)
            g = jnp.concatenate(
                [jnp.full(xt.shape, 0.7310586, f32), xts, g23], axis=0)

            CI = g.shape[0]
            padz = jnp.zeros((CI, 2 * W), dtype=g.dtype)
            gext = jnp.concatenate([padz, g, padz], axis=1).astype(jnp.bfloat16)
            gl = jnp.where(lane320 != W - 1, gext, jnp.bfloat16(0))
            gr = jnp.where(lane320 != 0, gext, jnp.bfloat16(0))

            acck = jnp.zeros((o_ref.shape[1], HW), f32)
            for j in range(9):
                dy, dx = j // 3, j % 3
                off = W * (dy - 1) + (dx - 1)
                src = (gl, gext, gr)[dx]
                s = jax.lax.slice(src, (0, 2 * W + off),
                                  (CI, 2 * W + off + HW))
                acck = acck + jax.lax.dot(
                    w_ref[e, j], s, preferred_element_type=f32)
            acc = acc + gate * acck

        o_ref[b] = acc
        return carry

    jax.lax.fori_loop(0, B, sample, 0)


def kernel(x, w_gate, poly_weights, beta_weights):
    B, C, H, W = x.shape
    E, O, CI, KH, KW = poly_weights.shape
    HW = H * W
    x2 = x.reshape(B, C, HW)
    pwt = jnp.transpose(poly_weights.astype(jnp.bfloat16),
                        (0, 3, 4, 1, 2)).reshape(E, KH * KW, O, CI)

    y, loss = pl.pallas_call(
        _fused_body,
        out_shape=[
            jax.ShapeDtypeStruct((B, O, HW), jnp.float32),
            jax.ShapeDtypeStruct((1, 1), jnp.float32),
        ],
    )(x2, w_gate, pwt, beta_weights)

    return y.reshape(B, O, H, W), loss[0, 0]


# speedup vs baseline: 1.0745x; 1.0646x over previous
import jax
import jax.numpy as jnp
from jax.experimental import pallas as pl
from jax.experimental.pallas import tpu as pltpu

_K = 2


def _fused_body(x_ref, wg_ref, w_ref, beta_ref, o_ref, loss_ref,
                xt_s, xts_s):
    B = x_ref.shape[0]
    E = wg_ref.shape[1]
    f32 = jnp.float32

    xm = jnp.mean(x_ref[...], axis=2)
    logits = jnp.dot(xm, wg_ref[...], preferred_element_type=f32)
    m = jnp.max(logits, axis=1, keepdims=True)
    ex = jnp.exp(logits - m)
    sm = ex / jnp.sum(ex, axis=1, keepdims=True)

    col = jax.lax.broadcasted_iota(jnp.int32, (B, E), 1)
    v1 = jnp.max(sm, axis=1, keepdims=True)
    i1 = jnp.min(jnp.where(sm == v1, col, E + 1), axis=1, keepdims=True)
    sm2 = jnp.where(col == i1, -jnp.inf, sm)
    v2 = jnp.max(sm2, axis=1, keepdims=True)
    i2 = jnp.min(jnp.where(sm2 == v2, col, E + 1), axis=1, keepdims=True)

    den = v1 + v2 + 1e-6
    g1 = v1 / den
    g2 = v2 / den

    dense = jnp.where(col == i1, g1, 0.0) + jnp.where(col == i2, g2, 0.0)
    imp = jnp.sum(dense, axis=0)
    load = jnp.sum((dense > 0.0).astype(f32), axis=0)

    def cv_sq(v):
        mu = jnp.mean(v)
        var = jnp.sum((v - mu) ** 2) / (E - 1)
        return var / (mu * mu + 1e-10)

    loss_ref[...] = jnp.reshape((cv_sq(imp) + cv_sq(load)) * 1e-2, (1, 1))

    brow = jax.lax.broadcasted_iota(jnp.int32, (B, 1), 0)
    bv = beta_ref[...]
    ri = jax.lax.broadcasted_iota(jnp.int32, bv.shape, 0)
    ci_ = jax.lax.broadcasted_iota(jnp.int32, bv.shape, 1)
    W = 16
    HW = x_ref.shape[2]
    lane320 = jax.lax.broadcasted_iota(jnp.int32, (1, HW + 4 * W), 1) % W
    bf16 = jnp.bfloat16

    xtall = jnp.tanh(x_ref[...])
    xt_s[...] = xtall
    xts_s[...] = (xtall * jax.nn.sigmoid(xtall)).astype(bf16)

    C = x_ref.shape[1]
    cb0 = jnp.full((C, HW), 0.7310586, bf16)

    def sample(b, carry):
        xt = xt_s[b]
        xt2 = xt * xt
        xts = xts_s[b]

        def poly23(e_):
            b2 = 2.25 * jnp.sum(jnp.where((ri == e_) & (ci_ == 1), bv, 0.0))
            b3 = (300.0 / 9.0) * jnp.sum(
                jnp.where((ri == e_) & (ci_ == 2), bv, 0.0))
            p2 = xt2 - b2
            p3 = xt * p2 - b3 * xt
            return p2, p3

        e0 = jnp.sum(jnp.where(brow == b, i1, 0))
        e1 = jnp.sum(jnp.where(brow == b, i2, 0))
        gate0 = jnp.sum(jnp.where(brow == b, g1, 0.0))
        gate1 = jnp.sum(jnp.where(brow == b, g2, 0.0))
        p2a, p3a = poly23(e0)
        p2b, p3b = poly23(e1)
        pp = jnp.concatenate([p2a, p3a, p2b, p3b], axis=0)
        pp = (pp * jax.nn.sigmoid(pp)).astype(bf16)

        acc = jnp.zeros((o_ref.shape[1], HW), f32)
        for k in range(_K):
            e = e0 if k == 0 else e1
            gate = gate0 if k == 0 else gate1
            s23 = jax.lax.slice(pp, (2 * C * k, 0), (2 * C * (k + 1), HW))
            g = jnp.concatenate([cb0, xts, s23], axis=0)

            CI = g.shape[0]
            padz = jnp.zeros((CI, 2 * W), dtype=bf16)
            gext = jnp.concatenate([padz, g, padz], axis=1)
            gl = jnp.where(lane320 != W - 1, gext, bf16(0))
            gr = jnp.where(lane320 != 0, gext, bf16(0))

            acck = jnp.zeros((o_ref.shape[1], HW), f32)
            acck2 = jnp.zeros((o_ref.shape[1], HW), f32)
            for j in range(9):
                dy, dx = j // 3, j % 3
                off = W * (dy - 1) + (dx - 1)
                src = (gl, gext, gr)[dx]
                s = jax.lax.slice(src, (0, 2 * W + off),
                                  (CI, 2 * W + off + HW))
                d = jax.lax.dot(w_ref[e, j], s, preferred_element_type=f32)
                if j % 2 == 0:
                    acck = acck + d
                else:
                    acck2 = acck2 + d
            acc = acc + gate * (acck + acck2)

        o_ref[b] = acc
        return carry

    jax.lax.fori_loop(0, B, sample, 0)


def kernel(x, w_gate, poly_weights, beta_weights):
    B, C, H, W = x.shape
    E, O, CI, KH, KW = poly_weights.shape
    HW = H * W
    x2 = x.reshape(B, C, HW)
    pwt = jnp.transpose(poly_weights.astype(jnp.bfloat16),
                        (0, 3, 4, 1, 2)).reshape(E, KH * KW, O, CI)

    y, loss = pl.pallas_call(
        _fused_body,
        out_shape=[
            jax.ShapeDtypeStruct((B, O, HW), jnp.float32),
            jax.ShapeDtypeStruct((1, 1), jnp.float32),
        ],
        scratch_shapes=[
            pltpu.VMEM((B, C, HW), jnp.float32),
            pltpu.VMEM((B, C, HW), jnp.bfloat16),
        ],
    )(x2, w_gate, pwt, beta_weights)

    return y.reshape(B, O, H, W), loss[0, 0]


# speedup vs baseline: 1.2671x; 1.1793x over previous
import jax
import jax.numpy as jnp
from jax.experimental import pallas as pl
from jax.experimental.pallas import tpu as pltpu

_K = 2


def _fused_body(x_ref, wg_ref, wf_ref, beta_ref, o_ref, loss_ref,
                xt_s, xts_s, w_ref):
    B = x_ref.shape[0]
    E = wg_ref.shape[1]
    f32 = jnp.float32

    xm = jnp.mean(x_ref[...], axis=2)
    logits = jnp.dot(xm, wg_ref[...], preferred_element_type=f32)
    m = jnp.max(logits, axis=1, keepdims=True)
    ex = jnp.exp(logits - m)
    sm = ex / jnp.sum(ex, axis=1, keepdims=True)

    col = jax.lax.broadcasted_iota(jnp.int32, (B, E), 1)
    v1 = jnp.max(sm, axis=1, keepdims=True)
    i1 = jnp.min(jnp.where(sm == v1, col, E + 1), axis=1, keepdims=True)
    sm2 = jnp.where(col == i1, -jnp.inf, sm)
    v2 = jnp.max(sm2, axis=1, keepdims=True)
    i2 = jnp.min(jnp.where(sm2 == v2, col, E + 1), axis=1, keepdims=True)

    den = v1 + v2 + 1e-6
    g1 = v1 / den
    g2 = v2 / den

    dense = jnp.where(col == i1, g1, 0.0) + jnp.where(col == i2, g2, 0.0)
    imp = jnp.sum(dense, axis=0)
    load = jnp.sum((dense > 0.0).astype(f32), axis=0)

    def cv_sq(v):
        mu = jnp.mean(v)
        var = jnp.sum((v - mu) ** 2) / (E - 1)
        return var / (mu * mu + 1e-10)

    loss_ref[...] = jnp.reshape((cv_sq(imp) + cv_sq(load)) * 1e-2, (1, 1))

    brow = jax.lax.broadcasted_iota(jnp.int32, (B, 1), 0)
    bv = beta_ref[...]
    ri = jax.lax.broadcasted_iota(jnp.int32, bv.shape, 0)
    ci_ = jax.lax.broadcasted_iota(jnp.int32, bv.shape, 1)
    W = 16
    HW = x_ref.shape[2]
    lane320 = jax.lax.broadcasted_iota(jnp.int32, (1, HW + 4 * W), 1) % W
    bf16 = jnp.bfloat16

    xtall = jnp.tanh(x_ref[...])
    xt_s[...] = xtall
    xts_s[...] = (xtall * jax.nn.sigmoid(xtall)).astype(bf16)
    w_ref[...] = wf_ref[...].astype(bf16)

    C = x_ref.shape[1]
    cb0 = jnp.full((C, HW), 0.7310586, bf16)

    def sample(b, carry):
        xt = xt_s[b]
        xt2 = xt * xt
        xts = xts_s[b]

        def poly23(e_):
            b2 = 2.25 * jnp.sum(jnp.where((ri == e_) & (ci_ == 1), bv, 0.0))
            b3 = (300.0 / 9.0) * jnp.sum(
                jnp.where((ri == e_) & (ci_ == 2), bv, 0.0))
            p2 = xt2 - b2
            p3 = xt * p2 - b3 * xt
            return p2, p3

        e0 = jnp.sum(jnp.where(brow == b, i1, 0))
        e1 = jnp.sum(jnp.where(brow == b, i2, 0))
        gate0 = jnp.sum(jnp.where(brow == b, g1, 0.0))
        gate1 = jnp.sum(jnp.where(brow == b, g2, 0.0))
        p2a, p3a = poly23(e0)
        p2b, p3b = poly23(e1)
        pp = jnp.concatenate([p2a, p3a, p2b, p3b], axis=0)
        pp = (pp * jax.nn.sigmoid(pp)).astype(bf16)

        acc = jnp.zeros((o_ref.shape[1], HW), f32)
        for k in range(_K):
            e = e0 if k == 0 else e1
            gate = gate0 if k == 0 else gate1
            s23 = jax.lax.slice(pp, (2 * C * k, 0), (2 * C * (k + 1), HW))
            g = jnp.concatenate([cb0, xts, s23], axis=0)

            CI = g.shape[0]
            padz = jnp.zeros((CI, 2 * W), dtype=bf16)
            gext = jnp.concatenate([padz, g, padz], axis=1)
            gl = jnp.where(lane320 != W - 1, gext, bf16(0))
            gr = jnp.where(lane320 != 0, gext, bf16(0))

            acck = jnp.zeros((o_ref.shape[1], HW), f32)
            acck2 = jnp.zeros((o_ref.shape[1], HW), f32)
            for j in range(9):
                dy, dx = j // 3, j % 3
                off = W * (dy - 1) + (dx - 1)
                src = (gl, gext, gr)[dx]
                s = jax.lax.slice(src, (0, 2 * W + off),
                                  (CI, 2 * W + off + HW))
                d = jax.lax.dot(w_ref[e, j], s, preferred_element_type=f32)
                if j % 2 == 0:
                    acck = acck + d
                else:
                    acck2 = acck2 + d
            acc = acc + gate * (acck + acck2)

        o_ref[b] = acc
        return carry

    jax.lax.fori_loop(0, B, sample, 0)


def kernel(x, w_gate, poly_weights, beta_weights):
    B, C, H, W = x.shape
    E, O, CI, KH, KW = poly_weights.shape
    HW = H * W
    x2 = x.reshape(B, C, HW)
    pwt = jnp.transpose(poly_weights, (0, 3, 4, 1, 2)).reshape(
        E, KH * KW, O, CI)

    y, loss = pl.pallas_call(
        _fused_body,
        out_shape=[
            jax.ShapeDtypeStruct((B, O, HW), jnp.float32),
            jax.ShapeDtypeStruct((1, 1), jnp.float32),
        ],
        scratch_shapes=[
            pltpu.VMEM((B, C, HW), jnp.float32),
            pltpu.VMEM((B, C, HW), jnp.bfloat16),
            pltpu.VMEM((E, KH * KW, O, CI), jnp.bfloat16),
        ],
    )(x2, w_gate, pwt, beta_weights)

    return y.reshape(B, O, H, W), loss[0, 0]
